# Initial kernel scaffold; baseline (speedup 1.0000x reference)
import jax
import jax.numpy as jnp
from jax.experimental import pallas as pl
from jax.experimental.pallas import tpu as pltpu

_N = 10000
_F = 128
_H = 128
_C = 10
_K = 8
_ALPHA = 0.1

_BM = 400


def _encoder_kernel(x_ref, w1_ref, b1_ref, w2_ref, b2_ref, z_ref):
    h = jax.lax.dot_general(
        x_ref[...], w1_ref[...], (((1,), (1,)), ((), ())),
        preferred_element_type=jnp.float32)
    h = jax.nn.relu(h + b1_ref[...])
    z = jax.lax.dot_general(
        h, w2_ref[...], (((1,), (1,)), ((), ())),
        preferred_element_type=jnp.float32)
    z_ref[...] = z + b2_ref[...]


def _prop_kernel(adj_ref, z_full_ref, z_blk_ref, out_ref, cur_ref):
    k = pl.program_id(0)
    i = pl.program_id(1)
    prev = jnp.where(
        k == 0, z_full_ref[...], cur_ref[jnp.remainder(k, 2)])
    y = jax.lax.dot_general(
        adj_ref[...], prev, (((1,), (0,)), ((), ())),
        preferred_element_type=jnp.float32)
    y = y * (1.0 - _ALPHA)
    y = y + _ALPHA * z_blk_ref[...]
    cur_ref[jnp.remainder(k + 1, 2), pl.ds(i * _BM, _BM), :] = y

    @pl.when(k == _K - 1)
    def _():
        m = jnp.max(y, axis=1, keepdims=True)
        shifted = y - m
        lse = jnp.log(jnp.sum(jnp.exp(shifted), axis=1, keepdims=True))
        out_ref[...] = shifted - lse


def kernel(x, adj, W1, b1, W2, b2):
    z = pl.pallas_call(
        _encoder_kernel,
        grid=(_N // 1000,),
        in_specs=[
            pl.BlockSpec((1000, _F), lambda i: (i, 0)),
            pl.BlockSpec((_H, _F), lambda i: (0, 0)),
            pl.BlockSpec((1, _H), lambda i: (0, 0)),
            pl.BlockSpec((_C, _H), lambda i: (0, 0)),
            pl.BlockSpec((1, _C), lambda i: (0, 0)),
        ],
        out_specs=pl.BlockSpec((1000, _C), lambda i: (i, 0)),
        out_shape=jax.ShapeDtypeStruct((_N, _C), jnp.float32),
    )(x, W1, b1.reshape(1, _H), W2, b2.reshape(1, _C))

    out = pl.pallas_call(
        _prop_kernel,
        grid=(_K, _N // _BM),
        in_specs=[
            pl.BlockSpec((_BM, _N), lambda k, i: (i, 0)),
            pl.BlockSpec((_N, _C), lambda k, i: (0, 0)),
            pl.BlockSpec((_BM, _C), lambda k, i: (i, 0)),
        ],
        out_specs=pl.BlockSpec((_BM, _C), lambda k, i: (i, 0)),
        out_shape=jax.ShapeDtypeStruct((_N, _C), jnp.float32),
        scratch_shapes=[pltpu.VMEM((2, _N, _C), jnp.float32)],
    )(adj, z, z)
    return out


# speedup vs baseline: 1.1766x; 1.1766x over previous
import jax
import jax.numpy as jnp
from jax.experimental import pallas as pl
from jax.experimental.pallas import tpu as pltpu

_N = 10000
_F = 128
_H = 128
_C = 10
_K = 8
_ALPHA = 0.1

_BM = 400


def _encoder_kernel(x_ref, w1_ref, b1_ref, w2_ref, b2_ref, z_ref):
    h = jax.lax.dot_general(
        x_ref[...], w1_ref[...], (((1,), (1,)), ((), ())),
        preferred_element_type=jnp.float32)
    h = jax.nn.relu(h + b1_ref[...])
    z = jax.lax.dot_general(
        h, w2_ref[...], (((1,), (1,)), ((), ())),
        preferred_element_type=jnp.float32)
    z_ref[...] = z + b2_ref[...]


def _prop_kernel(adj_ref, z_full_ref, out_ref, cur_ref):
    k = pl.program_id(0)
    i = pl.program_id(1)
    prev = jnp.where(
        k == 0, z_full_ref[...], cur_ref[jnp.remainder(k, 2)])
    y = jax.lax.dot_general(
        adj_ref[...], prev, (((1,), (0,)), ((), ())),
        preferred_element_type=jnp.float32)
    y = y * (1.0 - _ALPHA)
    y = y + _ALPHA * z_full_ref[pl.ds(i * _BM, _BM), :]
    cur_ref[jnp.remainder(k + 1, 2), pl.ds(i * _BM, _BM), :] = y

    @pl.when(k == _K - 1)
    def _():
        m = jnp.max(y, axis=1, keepdims=True)
        shifted = y - m
        lse = jnp.log(jnp.sum(jnp.exp(shifted), axis=1, keepdims=True))
        out_ref[pl.ds(i * _BM, _BM), :] = shifted - lse


def kernel(x, adj, W1, b1, W2, b2):
    z = pl.pallas_call(
        _encoder_kernel,
        grid=(_N // 1000,),
        in_specs=[
            pl.BlockSpec((1000, _F), lambda i: (i, 0)),
            pl.BlockSpec((_H, _F), lambda i: (0, 0)),
            pl.BlockSpec((1, _H), lambda i: (0, 0)),
            pl.BlockSpec((_C, _H), lambda i: (0, 0)),
            pl.BlockSpec((1, _C), lambda i: (0, 0)),
        ],
        out_specs=pl.BlockSpec((1000, _C), lambda i: (i, 0)),
        out_shape=jax.ShapeDtypeStruct((_N, _C), jnp.float32),
    )(x, W1, b1.reshape(1, _H), W2, b2.reshape(1, _C))

    out = pl.pallas_call(
        _prop_kernel,
        grid=(_K, _N // _BM),
        in_specs=[
            pl.BlockSpec((_BM, _N), lambda k, i: (i, 0)),
            pl.BlockSpec((_N, _C), lambda k, i: (0, 0)),
        ],
        out_specs=pl.BlockSpec((_N, _C), lambda k, i: (0, 0)),
        out_shape=jax.ShapeDtypeStruct((_N, _C), jnp.float32),
        scratch_shapes=[pltpu.VMEM((2, _N, _C), jnp.float32)],
    )(adj, z)
    return out


# speedup vs baseline: 1.8967x; 1.6120x over previous
import jax
import jax.numpy as jnp
from jax.experimental import pallas as pl
from jax.experimental.pallas import tpu as pltpu

_N = 10000
_F = 128
_H = 128
_C = 10
_K = 8
_ALPHA = 0.1

_BM = 400


def _encoder_kernel(x_ref, w1_ref, b1_ref, w2_ref, b2_ref, z_ref):
    h = jax.lax.dot_general(
        x_ref[...], w1_ref[...], (((1,), (1,)), ((), ())),
        preferred_element_type=jnp.float32)
    h = jax.nn.relu(h + b1_ref[...])
    z = jax.lax.dot_general(
        h, w2_ref[...], (((1,), (1,)), ((), ())),
        preferred_element_type=jnp.float32)
    z_ref[...] = z + b2_ref[...]


def _quant_step0_kernel(adj_ref, z_ref, adj8_ref, cur1_ref):
    i = pl.program_id(0)
    a8 = adj_ref[...].astype(jnp.float8_e4m3fn)
    adj8_ref[...] = a8
    zb = z_ref[...].astype(jnp.bfloat16)
    y = jax.lax.dot_general(
        a8.astype(jnp.bfloat16), zb, (((1,), (0,)), ((), ())),
        preferred_element_type=jnp.float32)
    y = y * (1.0 - _ALPHA)
    y = y + _ALPHA * z_ref[pl.ds(i * _BM, _BM), :]
    cur1_ref[...] = y


def _prop_kernel(adj8_ref, z_ref, cur1_ref, out_ref, cur_ref):
    k = pl.program_id(0)
    i = pl.program_id(1)
    prev = jnp.where(
        k == 0, cur1_ref[...], cur_ref[jnp.remainder(k, 2)])
    y = jax.lax.dot_general(
        adj8_ref[...].astype(jnp.bfloat16), prev.astype(jnp.bfloat16),
        (((1,), (0,)), ((), ())),
        preferred_element_type=jnp.float32)
    y = y * (1.0 - _ALPHA)
    y = y + _ALPHA * z_ref[pl.ds(i * _BM, _BM), :]
    cur_ref[jnp.remainder(k + 1, 2), pl.ds(i * _BM, _BM), :] = y

    @pl.when(k == _K - 2)
    def _():
        m = jnp.max(y, axis=1, keepdims=True)
        shifted = y - m
        lse = jnp.log(jnp.sum(jnp.exp(shifted), axis=1, keepdims=True))
        out_ref[pl.ds(i * _BM, _BM), :] = shifted - lse


def kernel(x, adj, W1, b1, W2, b2):
    z = pl.pallas_call(
        _encoder_kernel,
        grid=(_N // 1000,),
        in_specs=[
            pl.BlockSpec((1000, _F), lambda i: (i, 0)),
            pl.BlockSpec((_H, _F), lambda i: (0, 0)),
            pl.BlockSpec((1, _H), lambda i: (0, 0)),
            pl.BlockSpec((_C, _H), lambda i: (0, 0)),
            pl.BlockSpec((1, _C), lambda i: (0, 0)),
        ],
        out_specs=pl.BlockSpec((1000, _C), lambda i: (i, 0)),
        out_shape=jax.ShapeDtypeStruct((_N, _C), jnp.float32),
    )(x, W1, b1.reshape(1, _H), W2, b2.reshape(1, _C))

    adj8, cur1 = pl.pallas_call(
        _quant_step0_kernel,
        grid=(_N // _BM,),
        in_specs=[
            pl.BlockSpec((_BM, _N), lambda i: (i, 0)),
            pl.BlockSpec((_N, _C), lambda i: (0, 0)),
        ],
        out_specs=[
            pl.BlockSpec((_BM, _N), lambda i: (i, 0)),
            pl.BlockSpec((_BM, _C), lambda i: (i, 0)),
        ],
        out_shape=[
            jax.ShapeDtypeStruct((_N, _N), jnp.float8_e4m3fn),
            jax.ShapeDtypeStruct((_N, _C), jnp.float32),
        ],
    )(adj, z)

    out = pl.pallas_call(
        _prop_kernel,
        grid=(_K - 1, _N // _BM),
        in_specs=[
            pl.BlockSpec((_BM, _N), lambda k, i: (i, 0)),
            pl.BlockSpec((_N, _C), lambda k, i: (0, 0)),
            pl.BlockSpec((_N, _C), lambda k, i: (0, 0)),
        ],
        out_specs=pl.BlockSpec((_N, _C), lambda k, i: (0, 0)),
        out_shape=jax.ShapeDtypeStruct((_N, _C), jnp.float32),
        scratch_shapes=[pltpu.VMEM((2, _N, _C), jnp.float32)],
    )(adj8, z, cur1)
    return out
